# baseline (speedup 1.0000x reference)
import functools

import jax
import jax.numpy as jnp
import numpy as np
from jax import lax
from jax.experimental import pallas as pl
from jax.experimental.pallas import tpu as pltpu
from jax.experimental.pallas import tpu_sc as plsc

N = 10000
E = 320000
D = 128
K = 5000

NPAD = 10240
HN = 16384
SROW = 128
NC = 2
NS = 16
NW = NC * NS
ECH = 80
EW = NW * ECH * 128 - 0
RPW = NPAD // NW
SEG = 16384
KPAD = 5120
_GRP = 16

_HALF = [10080] * 11 + [9840] * 4 + [9760]
_BOUNDS = []
_b = 0
for _h in range(2):
    for _s in _HALF:
        _b += _s
        if _b < E:
            _BOUNDS.append(_b)
NSPL = len(_BOUNDS)


def _kdeg_body(dst_hbm, ones_hbm, zeros_hbm, hist_hbm,
               dst_v, ones_v, hist_sh, sem_s):
    cid = lax.axis_index("c")
    sid = lax.axis_index("s")
    w = cid * NS + sid

    @pl.when(sid == 0)
    def _():
        pltpu.sync_copy(zeros_hbm, hist_sh)

    pltpu.sync_copy(dst_hbm.at[w], dst_v)
    pltpu.sync_copy(ones_hbm, ones_v)
    plsc.subcore_barrier()

    def fire_s(c, cr):
        pltpu.async_copy(ones_v.at[c], hist_sh.at[dst_v.at[c]], sem_s,
                         add=True)
        return cr

    def drain_s(c, cr):
        pltpu.make_async_copy(ones_v.at[c], hist_sh.at[dst_v.at[c]],
                              sem_s).wait()
        return cr

    for g in range(ECH // _GRP):
        lax.fori_loop(g * _GRP, (g + 1) * _GRP, fire_s, 0)
        lax.fori_loop(g * _GRP, (g + 1) * _GRP, drain_s, 0)

    plsc.subcore_barrier()

    @pl.when(sid == 0)
    def _():
        pltpu.sync_copy(hist_sh, hist_hbm.at[cid])


def _kdeg(dst3, ones2, zeros_i):
    mesh = plsc.VectorSubcoreMesh(core_axis_name="c", subcore_axis_name="s")
    return pl.kernel(
        _kdeg_body,
        out_type=jax.ShapeDtypeStruct((NC, HN), jnp.int32),
        mesh=mesh,
        scratch_types=[
            pltpu.VMEM((ECH, 128), jnp.int32),
            pltpu.VMEM((ECH, 128), jnp.int32),
            pltpu.VMEM_SHARED((HN,), jnp.int32),
            pltpu.SemaphoreType.DMA,
        ],
    )(dst3, ones2, zeros_i)


def _sread(ref, r, k):
    return ref[r, pl.ds(k * 16, 16)][0]


def _kcum_body(deg_ref, spl_ref):
    deg = (deg_ref[0] + deg_ref[1]).astype(jnp.float32)
    hi = jax.lax.Precision.HIGHEST
    r128 = lax.broadcasted_iota(jnp.int32, (128, 128), 0)
    c128 = lax.broadcasted_iota(jnp.int32, (128, 128), 1)
    upper = (r128 < c128).astype(jnp.float32)
    inrow = jnp.dot(deg, upper, precision=hi)
    ones_col = jnp.ones((128, 1), jnp.float32)
    rowsum = jnp.dot(deg, ones_col, precision=hi)
    r80 = lax.broadcasted_iota(jnp.int32, (80, 80), 0)
    c80 = lax.broadcasted_iota(jnp.int32, (80, 80), 1)
    lower = (c80 < r80).astype(jnp.float32)
    base = jnp.dot(lower, rowsum, precision=hi)
    cum = inrow + base

    p0 = lax.broadcasted_iota(jnp.int32, (80, 128), 0)
    p1 = lax.broadcasted_iota(jnp.int32, (80, 128), 1)
    pos2d = p0 * 128 + p1

    ii0 = lax.broadcasted_iota(jnp.int32, (8, 512), 0)
    ii1 = lax.broadcasted_iota(jnp.int32, (8, 512), 1)
    out = jnp.zeros((8, 512), jnp.int32)
    for k, b in enumerate(_BOUNDS):
        bf = jnp.float32(b)
        le = cum <= bf
        row = jnp.sum(le.astype(jnp.float32)).astype(jnp.int32) - 1
        cumval = jnp.max(jnp.where(le, cum, -1.0))
        off = (bf - cumval).astype(jnp.int32)
        own = row // RPW
        ws = jnp.sum(jnp.where(pos2d == own * RPW, cum, 0.0))
        spos = own * SEG + (cumval - ws).astype(jnp.int32)
        dgk = jnp.sum(jnp.where(pos2d == row, deg, 0.0)).astype(jnp.int32)
        out = out + jnp.where((ii0 == 0) & (ii1 == 16 * k), row, 0)
        out = out + jnp.where((ii0 == 1) & (ii1 == 16 * k), off, 0)
        out = out + jnp.where((ii0 == 2) & (ii1 == 16 * k), spos, 0)
        out = out + jnp.where((ii0 == 3) & (ii1 == 16 * k), dgk, 0)
    for w in range(NW):
        ws = jnp.sum(jnp.where(pos2d == w * RPW, cum, 0.0))
        we = (jnp.sum(jnp.where(pos2d == (w + 1) * RPW, cum, 0.0))
              if w < NW - 1 else jnp.float32(E))
        out = out + jnp.where((ii0 == 4) & (ii1 == 16 * w),
                              (we - ws).astype(jnp.int32), 0)
    spl_ref[...] = out


EROWS = 2560


def _kpos_body(dst_ref, gpos_ref, run_ref):
    g = pl.program_id(0)

    @pl.when(g == 0)
    def _():
        run_ref[...] = jnp.zeros((8, 128), jnp.float32)

    hi = jax.lax.Precision.HIGHEST
    own = dst_ref[...] // RPW
    r128 = lax.broadcasted_iota(jnp.int32, (128, 128), 0)
    c128 = lax.broadcasted_iota(jnp.int32, (128, 128), 1)
    upper = (r128 < c128).astype(jnp.float32)
    lower = (r128 > c128).astype(jnp.float32)
    ones_row = jnp.ones((1, 128), jnp.float32)
    run = run_ref[0:1, :]

    gpos = jnp.zeros((128, 128), jnp.float32)
    newrun = jnp.zeros((1, 128), jnp.float32)
    for w in range(NW + 1):
        mw = (own == w).astype(jnp.float32)
        inrow = jnp.dot(mw, upper, precision=hi)
        rc = jnp.sum(mw, axis=1, keepdims=True)
        wgp = jnp.dot(lower, rc * ones_row, precision=hi)[:, 0:1]
        rank = inrow + wgp + run[0:1, w:w + 1]
        gpos = gpos + mw * (jnp.float32(w * SEG) + rank)
        gs = jnp.sum(rc)
        cidx = lax.broadcasted_iota(jnp.int32, (1, 128), 1)
        newrun = newrun + jnp.where(cidx == w, gs, 0.0)
    run_ref[0:1, :] = run + newrun
    gpos_ref[...] = gpos.astype(jnp.int32)


def _kpos(dst2d):
    return pl.pallas_call(
        _kpos_body,
        grid=(EROWS // 128,),
        in_specs=[pl.BlockSpec((128, 128), lambda g: (g, 0))],
        out_specs=pl.BlockSpec((128, 128), lambda g: (g, 0)),
        out_shape=jax.ShapeDtypeStruct((EROWS, 128), jnp.int32),
        scratch_shapes=[pltpu.VMEM((8, 128), jnp.float32)],
    )(dst2d)


def _kroute_body(src_hbm, dst_hbm, gpos_hbm, csrs_hbm, csrd_hbm,
                 src_v, dst_v, gpos_v, sem):
    cid = lax.axis_index("c")
    sid = lax.axis_index("s")
    w = cid * NS + sid
    pltpu.sync_copy(src_hbm.at[w], src_v)
    pltpu.sync_copy(dst_hbm.at[w], dst_v)
    pltpu.sync_copy(gpos_hbm.at[w], gpos_v)

    def fire(c, cr):
        pltpu.async_copy(src_v.at[c], csrs_hbm.at[gpos_v.at[c]], sem)
        pltpu.async_copy(dst_v.at[c], csrd_hbm.at[gpos_v.at[c]], sem)
        return cr

    def drain(c, cr):
        pltpu.make_async_copy(src_v.at[c], csrs_hbm.at[gpos_v.at[c]],
                              sem).wait()
        pltpu.make_async_copy(dst_v.at[c], csrd_hbm.at[gpos_v.at[c]],
                              sem).wait()
        return cr

    for g in range(ECH // _GRP):
        lax.fori_loop(g * _GRP, (g + 1) * _GRP, fire, 0)
        lax.fori_loop(g * _GRP, (g + 1) * _GRP, drain, 0)


def _kroute(src3, dst3, gpos3):
    mesh = plsc.VectorSubcoreMesh(core_axis_name="c", subcore_axis_name="s")
    return pl.kernel(
        _kroute_body,
        out_type=[
            jax.ShapeDtypeStruct(((NW + 1) * SEG,), jnp.int32),
            jax.ShapeDtypeStruct(((NW + 1) * SEG,), jnp.int32),
        ],
        mesh=mesh,
        scratch_types=[
            pltpu.VMEM((ECH, 128), jnp.int32),
            pltpu.VMEM((ECH, 128), jnp.int32),
            pltpu.VMEM((ECH, 128), jnp.int32),
            pltpu.SemaphoreType.DMA,
        ],
    )(src3, dst3, gpos3)


def _kcum(deg3):
    return pl.pallas_call(
        _kcum_body,
        in_specs=[pl.BlockSpec((NC, 80, 128), lambda: (0, 0, 0))],
        out_specs=pl.BlockSpec((8, 512), lambda: (0, 0)),
        out_shape=jax.ShapeDtypeStruct((8, 512), jnp.int32),
    )(deg3)


def _kmain_body(x_hbm, csrs_hbm, csrd_hbm, zrow_hbm, spl_hbm, agg_hbm,
                sidx, didx, rv, acc, spl_s, sem):
    cid = lax.axis_index("c")
    sid = lax.axis_index("s")
    w = cid * NS + sid
    lo = w * RPW
    base = w * SEG

    pltpu.sync_copy(zrow_hbm, acc)
    pltpu.sync_copy(spl_hbm, spl_s)
    cnt = _sread(spl_s, 4, w)

    def edge(j, c128_):
        r = didx[pl.ds(j, 16)][0] - lo
        for l in range(8):
            sl = pl.ds(l * 16, 16)
            acc[r, sl] = acc[r, sl] + rv[j, sl]
        return c128_

    def chunk(c, _):
        pltpu.sync_copy(csrs_hbm.at[pl.ds(base + c * 128, 128)], sidx)
        pltpu.sync_copy(csrd_hbm.at[pl.ds(base + c * 128, 144)], didx)
        for l in range(8):
            sl = pl.ds(l * 16, 16)
            v = sidx[sl]
            sidx[sl] = jnp.minimum(jnp.maximum(v, 0), N - 1)
        pltpu.async_copy(x_hbm.at[sidx], rv, sem).wait()
        n = jnp.minimum(cnt - c * 128, 128)
        lax.fori_loop(0, n, edge, c * 128)
        return _

    lax.fori_loop(0, (cnt + 127) // 128, chunk, 0)

    pltpu.sync_copy(acc, agg_hbm.at[pl.ds(lo, RPW)])


def _kmain(x, csrs, csrd, zrow, spl):
    mesh = plsc.VectorSubcoreMesh(core_axis_name="c", subcore_axis_name="s")
    return pl.kernel(
        _kmain_body,
        out_type=jax.ShapeDtypeStruct((NPAD, D), jnp.float32),
        mesh=mesh,
        scratch_types=[
            pltpu.VMEM((128,), jnp.int32),
            pltpu.VMEM((144,), jnp.int32),
            pltpu.VMEM((128, D), jnp.float32),
            pltpu.VMEM((RPW, D), jnp.float32),
            pltpu.VMEM((8, 512), jnp.int32),
            pltpu.SemaphoreType.DMA,
        ],
    )(x, csrs, csrd, zrow, spl)


def _kfix_body(x_hbm, csrs_hbm, csrd_hbm, spl_hbm, fix_hbm,
               sidx, didx, rv, res_v, cur_v, spl_s, sem):
    cid = lax.axis_index("c")
    sid = lax.axis_index("s")
    w = cid * NS + sid

    pltpu.sync_copy(spl_hbm, spl_s)
    rid = _sread(spl_s, 0, w)
    prev = _sread(spl_s, 0, jnp.maximum(w - 1, 0))
    active = (w < NSPL) & (rid >= 0) & ((w == 0) | (rid != prev))

    @pl.when(active)
    def _():
        own = rid // RPW
        base = own * SEG
        cntw = _sread(spl_s, 4, own)
        for l in range(8):
            sl = pl.ds(l * 16, 16)
            res_v[sl] = jnp.zeros((16,), jnp.float32)
            cur_v[sl] = jnp.zeros((16,), jnp.float32)

        def edge(j, m):
            mine = didx[pl.ds(j, 16)][0] == rid

            @pl.when(mine)
            def _():
                flush = m < -1
                for k2 in range(NSPL):
                    flush = flush | (
                        (spl_s[0, pl.ds(16 * k2, 16)][0] == rid)
                        & (spl_s[1, pl.ds(16 * k2, 16)][0] == m))

                @pl.when(flush)
                def _():
                    for l in range(8):
                        sl = pl.ds(l * 16, 16)
                        res_v[sl] = res_v[sl] + cur_v[sl]
                        cur_v[sl] = jnp.zeros((16,), jnp.float32)

                for l in range(8):
                    sl = pl.ds(l * 16, 16)
                    cur_v[sl] = cur_v[sl] + rv[j, sl]

            return m + jnp.where(mine, 1, 0)

        def chunk(c, m):
            pltpu.sync_copy(csrs_hbm.at[pl.ds(base + c * 128, 128)], sidx)
            pltpu.sync_copy(csrd_hbm.at[pl.ds(base + c * 128, 144)], didx)
            for l in range(8):
                sl = pl.ds(l * 16, 16)
                v = sidx[sl]
                sidx[sl] = jnp.minimum(jnp.maximum(v, 0), N - 1)
            pltpu.async_copy(x_hbm.at[sidx], rv, sem).wait()
            n = jnp.minimum(cntw - c * 128, 128)
            return lax.fori_loop(0, n, edge, m)

        lax.fori_loop(0, (cntw + 127) // 128, chunk, 0)

        for l in range(8):
            sl = pl.ds(l * 16, 16)
            res_v[sl] = res_v[sl] + cur_v[sl]
        pltpu.sync_copy(res_v, fix_hbm.at[w])


def _kfix(x, csrs, csrd, spl):
    mesh = plsc.VectorSubcoreMesh(core_axis_name="c", subcore_axis_name="s")
    return pl.kernel(
        _kfix_body,
        out_type=jax.ShapeDtypeStruct((NW, D), jnp.float32),
        mesh=mesh,
        scratch_types=[
            pltpu.VMEM((128,), jnp.int32),
            pltpu.VMEM((144,), jnp.int32),
            pltpu.VMEM((128, D), jnp.float32),
            pltpu.VMEM((D,), jnp.float32),
            pltpu.VMEM((D,), jnp.float32),
            pltpu.VMEM((8, 512), jnp.int32),
            pltpu.SemaphoreType.DMA,
        ],
    )(x, csrs, csrd, spl)


def _kscore_body(agg_ref, x_ref, wrel_ref, wroot_ref, clo_ref, deg_ref,
                 scal_ref, spl_ref, fix_ref, score_ref):
    b_rel = scal_ref[0]
    w_close = scal_ref[1]
    w_deg = scal_ref[2]
    w_score = scal_ref[3]
    bias = scal_ref[4]

    d1 = jnp.dot(agg_ref[...], wrel_ref[...])
    d2 = jnp.dot(x_ref[...], wroot_ref[...])
    fixd = jnp.dot(fix_ref[...], wrel_ref[...])

    rowi = lax.broadcasted_iota(jnp.int32, (NPAD, 1), 0)
    for k in range(NSPL):
        rid = spl_ref[0, 16 * k]
        if k == 0:
            valid = rid >= 0
        else:
            valid = (rid >= 0) & (rid != spl_ref[0, 16 * (k - 1)])
        d1 = jnp.where((rowi == rid) & valid, fixd[k:k + 1, 0:1], d1)

    gnn = (d1 + b_rel) + d2
    s1 = jnp.maximum(gnn, 0.0)
    cent = (clo_ref[...] * w_close + deg_ref[...] * w_deg) + bias
    sc = jnp.maximum(s1 * w_score + cent, 0.0)
    sc = jnp.where(rowi < N, sc, 0.0)
    sc = jnp.where(sc == 0.0, 0.0, sc)
    score_ref[...] = sc


def _kscore(agg, x_pad, wrel, wroot, clo2, deg2, scals, spl, fixrows):
    return pl.pallas_call(
        _kscore_body,
        in_specs=[
            pl.BlockSpec((NPAD, D), lambda: (0, 0)),
            pl.BlockSpec((NPAD, D), lambda: (0, 0)),
            pl.BlockSpec((D, 1), lambda: (0, 0)),
            pl.BlockSpec((D, 1), lambda: (0, 0)),
            pl.BlockSpec((NPAD, 1), lambda: (0, 0)),
            pl.BlockSpec((NPAD, 1), lambda: (0, 0)),
            pl.BlockSpec(memory_space=pltpu.SMEM),
            pl.BlockSpec(memory_space=pltpu.SMEM),
            pl.BlockSpec((NW, D), lambda: (0, 0)),
        ],
        out_specs=pl.BlockSpec((NPAD, 1), lambda: (0, 0)),
        out_shape=jax.ShapeDtypeStruct((NPAD, 1), jnp.float32),
    )(agg, x_pad, wrel, wroot, clo2, deg2, scals, spl, fixrows)


def _ksort_body(score_ref, vals_ref, idx_ref):
    score = score_ref[...]
    row = lax.broadcasted_iota(jnp.int32, (SROW, 128), 0)
    col = lax.broadcasted_iota(jnp.int32, (SROW, 128), 1)
    pos = row * 128 + col

    key = lax.bitcast_convert_type(score, jnp.int32)
    idx = pos

    for lk in range(1, 15):
        k = 1 << lk
        asc = (pos & k) == 0
        for lj in range(lk - 1, -1, -1):
            j = 1 << lj
            if j >= 128:
                m = j // 128
                low = (row & m) == 0
                kp = jnp.where(low, pltpu.roll(key, SROW - m, 0),
                               pltpu.roll(key, m, 0))
                ip = jnp.where(low, pltpu.roll(idx, SROW - m, 0),
                               pltpu.roll(idx, m, 0))
            else:
                low = (col & j) == 0
                kp = jnp.where(low, pltpu.roll(key, 128 - j, 1),
                               pltpu.roll(key, j, 1))
                ip = jnp.where(low, pltpu.roll(idx, 128 - j, 1),
                               pltpu.roll(idx, j, 1))
            mp = (key > kp) | ((key == kp) & (idx < ip))
            take_mine = (mp == low) == asc
            key = jnp.where(take_mine, key, kp)
            idx = jnp.where(take_mine, idx, ip)

    vals_ref[...] = lax.bitcast_convert_type(key, jnp.float32)
    idx_ref[...] = idx


def _ksort(score2d):
    return pl.pallas_call(
        _ksort_body,
        in_specs=[pl.BlockSpec((SROW, 128), lambda: (0, 0))],
        out_specs=[
            pl.BlockSpec((SROW, 128), lambda: (0, 0)),
            pl.BlockSpec((SROW, 128), lambda: (0, 0)),
        ],
        out_shape=[
            jax.ShapeDtypeStruct((SROW, 128), jnp.float32),
            jax.ShapeDtypeStruct((SROW, 128), jnp.int32),
        ],
    )(score2d)


def _kgather_body(x_hbm, perm_hbm, rows_hbm, idx_v, rows_v):
    cid = lax.axis_index("c")
    sid = lax.axis_index("s")
    w = cid * NS + sid
    pltpu.sync_copy(perm_hbm.at[w], idx_v)
    for b in range(2):
        pltpu.sync_copy(x_hbm.at[idx_v.at[b]], rows_v.at[b])
    pltpu.sync_copy(rows_v, rows_hbm.at[pl.ds(2 * w, 2)])


def _kgather(x, perm3):
    mesh = plsc.VectorSubcoreMesh(core_axis_name="c", subcore_axis_name="s")
    return pl.kernel(
        _kgather_body,
        out_type=jax.ShapeDtypeStruct((KPAD // 80, 80, D), jnp.float32),
        mesh=mesh,
        scratch_types=[
            pltpu.VMEM((2, 80), jnp.int32),
            pltpu.VMEM((2, 80, D), jnp.float32),
        ],
    )(x, perm3)


def _kscale_body(rows_ref, vals_ref, o_ref):
    o_ref[...] = rows_ref[...] * vals_ref[...]


def _kscale(rows, vals_col):
    tile = 1024
    return pl.pallas_call(
        _kscale_body,
        grid=(KPAD // tile,),
        in_specs=[
            pl.BlockSpec((tile, D), lambda i: (i, 0)),
            pl.BlockSpec((tile, 1), lambda i: (i, 0)),
        ],
        out_specs=pl.BlockSpec((tile, D), lambda i: (i, 0)),
        out_shape=jax.ShapeDtypeStruct((KPAD, D), jnp.float32),
    )(rows, vals_col)


def kernel(x, edge_index, closeness, degree, W_rel, b_rel, W_root,
           w_close, w_deg, w_score, bias):
    src = edge_index[0]
    dst = edge_index[1]

    epad = NW * ECH * 128 - E
    dst_pad = jnp.concatenate([dst, jnp.full((epad,), NPAD, jnp.int32)])
    src_pad = jnp.concatenate([src, jnp.zeros((epad,), jnp.int32)])
    dst3 = dst_pad.reshape(NW, ECH, 128)
    src3 = src_pad.reshape(NW, ECH, 128)
    ones2 = jnp.ones((ECH, 128), jnp.int32)
    zeros_i = jnp.zeros((HN,), jnp.int32)
    hist = _kdeg(dst3, ones2, zeros_i)

    deg3 = hist[:, :NPAD].reshape(NC, 80, 128)
    spl = _kcum(deg3)

    gpos = _kpos(dst_pad.reshape(EROWS, 128))
    csrs, csrd = _kroute(src3, dst3, gpos.reshape(NW, ECH, 128))

    zrow = jnp.zeros((RPW, D), jnp.float32)
    agg = _kmain(x, csrs, csrd, zrow, spl)
    fixrows = _kfix(x, csrs, csrd, spl)

    x_pad = jnp.concatenate([x, jnp.zeros((NPAD - N, D), jnp.float32)])

    def pad_col(a):
        return jnp.concatenate(
            [a, jnp.zeros((NPAD - N,), jnp.float32)]).reshape(NPAD, 1)

    scals = jnp.stack([b_rel[0], w_close[0], w_deg[0], w_score[0], bias[0]])
    score = _kscore(agg, x_pad, W_rel, W_root, pad_col(closeness),
                    pad_col(degree), scals, spl, fixrows)

    score2d = jnp.concatenate(
        [score.reshape(-1),
         jnp.zeros((SROW * 128 - NPAD,), jnp.float32)]).reshape(SROW, 128)
    top_vals2, perm2 = _ksort(score2d)

    perm_flat = perm2.reshape(-1)
    vals_flat = top_vals2.reshape(-1)
    perm3 = perm_flat[:KPAD].reshape(NW, 2, 80)
    rows3 = _kgather(x, perm3)

    rows = rows3.reshape(KPAD, D)
    vals_col = vals_flat[:KPAD].reshape(KPAD, 1)
    x_out = _kscale(rows, vals_col)[:K]

    perm = perm_flat[:K]
    batch_out = jnp.zeros((K,), jnp.int32)
    return (x_out, perm, batch_out)


# speedup vs baseline: 1.0919x; 1.0919x over previous
import functools

import jax
import jax.numpy as jnp
import numpy as np
from jax import lax
from jax.experimental import pallas as pl
from jax.experimental.pallas import tpu as pltpu
from jax.experimental.pallas import tpu_sc as plsc

N = 10000
E = 320000
D = 128
K = 5000

NPAD = 10240
HN = 16384
SROW = 128
NC = 2
NS = 16
NW = NC * NS
ECH = 80
EW = NW * ECH * 128 - 0
RPW = NPAD // NW
SEG = 16384
KPAD = 5120
_GRP = 16

_HALF = [10080] * 11 + [9840] * 4 + [9760]
_BOUNDS = []
_b = 0
for _h in range(2):
    for _s in _HALF:
        _b += _s
        if _b < E:
            _BOUNDS.append(_b)
NSPL = len(_BOUNDS)


def _kdeg_body(dst_hbm, ones_hbm, zeros_hbm, hist_hbm,
               dst_v, ones_v, hist_sh, sem_s):
    cid = lax.axis_index("c")
    sid = lax.axis_index("s")
    w = cid * NS + sid

    @pl.when(sid == 0)
    def _():
        pltpu.sync_copy(zeros_hbm, hist_sh)

    pltpu.sync_copy(dst_hbm.at[w], dst_v)
    pltpu.sync_copy(ones_hbm, ones_v)
    plsc.subcore_barrier()

    def fire_s(c, cr):
        pltpu.async_copy(ones_v.at[c], hist_sh.at[dst_v.at[c]], sem_s,
                         add=True)
        return cr

    def drain_s(c, cr):
        pltpu.make_async_copy(ones_v.at[c], hist_sh.at[dst_v.at[c]],
                              sem_s).wait()
        return cr

    for g in range(ECH // _GRP):
        lax.fori_loop(g * _GRP, (g + 1) * _GRP, fire_s, 0)
        lax.fori_loop(g * _GRP, (g + 1) * _GRP, drain_s, 0)

    plsc.subcore_barrier()

    @pl.when(sid == 0)
    def _():
        pltpu.sync_copy(hist_sh, hist_hbm.at[cid])


def _kdeg(dst3, ones2, zeros_i):
    mesh = plsc.VectorSubcoreMesh(core_axis_name="c", subcore_axis_name="s")
    return pl.kernel(
        _kdeg_body,
        out_type=jax.ShapeDtypeStruct((NC, HN), jnp.int32),
        mesh=mesh,
        scratch_types=[
            pltpu.VMEM((ECH, 128), jnp.int32),
            pltpu.VMEM((ECH, 128), jnp.int32),
            pltpu.VMEM_SHARED((HN,), jnp.int32),
            pltpu.SemaphoreType.DMA,
        ],
    )(dst3, ones2, zeros_i)


def _sread(ref, r, k):
    return ref[r, pl.ds(k * 16, 16)][0]


def _kcum_body(deg_ref, spl_ref):
    deg = (deg_ref[0] + deg_ref[1]).astype(jnp.float32)
    hi = jax.lax.Precision.HIGHEST
    r128 = lax.broadcasted_iota(jnp.int32, (128, 128), 0)
    c128 = lax.broadcasted_iota(jnp.int32, (128, 128), 1)
    upper = (r128 < c128).astype(jnp.float32)
    inrow = jnp.dot(deg, upper, precision=hi)
    ones_col = jnp.ones((128, 1), jnp.float32)
    rowsum = jnp.dot(deg, ones_col, precision=hi)
    r80 = lax.broadcasted_iota(jnp.int32, (80, 80), 0)
    c80 = lax.broadcasted_iota(jnp.int32, (80, 80), 1)
    lower = (c80 < r80).astype(jnp.float32)
    base = jnp.dot(lower, rowsum, precision=hi)
    cum = inrow + base

    p0 = lax.broadcasted_iota(jnp.int32, (80, 128), 0)
    p1 = lax.broadcasted_iota(jnp.int32, (80, 128), 1)
    pos2d = p0 * 128 + p1

    ii0 = lax.broadcasted_iota(jnp.int32, (8, 512), 0)
    ii1 = lax.broadcasted_iota(jnp.int32, (8, 512), 1)
    out = jnp.zeros((8, 512), jnp.int32)
    for k, b in enumerate(_BOUNDS):
        bf = jnp.float32(b)
        le = cum <= bf
        row = jnp.sum(le.astype(jnp.float32)).astype(jnp.int32) - 1
        cumval = jnp.max(jnp.where(le, cum, -1.0))
        off = (bf - cumval).astype(jnp.int32)
        own = row // RPW
        ws = jnp.sum(jnp.where(pos2d == own * RPW, cum, 0.0))
        spos = own * SEG + (cumval - ws).astype(jnp.int32)
        dgk = jnp.sum(jnp.where(pos2d == row, deg, 0.0)).astype(jnp.int32)
        out = out + jnp.where((ii0 == 0) & (ii1 == 16 * k), row, 0)
        out = out + jnp.where((ii0 == 1) & (ii1 == 16 * k), off, 0)
        out = out + jnp.where((ii0 == 2) & (ii1 == 16 * k), spos, 0)
        out = out + jnp.where((ii0 == 3) & (ii1 == 16 * k), dgk, 0)
    for w in range(NW):
        ws = jnp.sum(jnp.where(pos2d == w * RPW, cum, 0.0))
        we = (jnp.sum(jnp.where(pos2d == (w + 1) * RPW, cum, 0.0))
              if w < NW - 1 else jnp.float32(E))
        out = out + jnp.where((ii0 == 4) & (ii1 == 16 * w),
                              (we - ws).astype(jnp.int32), 0)
    spl_ref[...] = out


EROWS = 2560


def _kpos_body(dst_ref, gpos_ref, run_ref):
    g = pl.program_id(0)

    @pl.when(g == 0)
    def _():
        run_ref[...] = jnp.zeros((8, 128), jnp.float32)

    hi = jax.lax.Precision.HIGHEST
    own = dst_ref[...] // RPW
    r128 = lax.broadcasted_iota(jnp.int32, (128, 128), 0)
    c128 = lax.broadcasted_iota(jnp.int32, (128, 128), 1)
    upper = (r128 < c128).astype(jnp.float32)
    lower = (r128 > c128).astype(jnp.float32)
    ones_row = jnp.ones((1, 128), jnp.float32)
    run = run_ref[0:1, :]

    gpos = jnp.zeros((128, 128), jnp.float32)
    newrun = jnp.zeros((1, 128), jnp.float32)
    for w in range(NW + 1):
        mw = (own == w).astype(jnp.float32)
        inrow = jnp.dot(mw, upper, precision=hi)
        rc = jnp.sum(mw, axis=1, keepdims=True)
        wgp = jnp.dot(lower, rc * ones_row, precision=hi)[:, 0:1]
        rank = inrow + wgp + run[0:1, w:w + 1]
        gpos = gpos + mw * (jnp.float32(w * SEG) + rank)
        gs = jnp.sum(rc)
        cidx = lax.broadcasted_iota(jnp.int32, (1, 128), 1)
        newrun = newrun + jnp.where(cidx == w, gs, 0.0)
    run_ref[0:1, :] = run + newrun
    gpos_ref[...] = gpos.astype(jnp.int32)


def _kpos(dst2d):
    return pl.pallas_call(
        _kpos_body,
        grid=(EROWS // 128,),
        in_specs=[pl.BlockSpec((128, 128), lambda g: (g, 0))],
        out_specs=pl.BlockSpec((128, 128), lambda g: (g, 0)),
        out_shape=jax.ShapeDtypeStruct((EROWS, 128), jnp.int32),
        scratch_shapes=[pltpu.VMEM((8, 128), jnp.float32)],
    )(dst2d)


def _kroute_body(src_hbm, dst_hbm, gpos_hbm, csrs_hbm, csrd_hbm,
                 src_v, dst_v, gpos_v, sem):
    cid = lax.axis_index("c")
    sid = lax.axis_index("s")
    w = cid * NS + sid
    pltpu.sync_copy(src_hbm.at[w], src_v)
    pltpu.sync_copy(dst_hbm.at[w], dst_v)
    pltpu.sync_copy(gpos_hbm.at[w], gpos_v)

    def fire(c, cr):
        pltpu.async_copy(src_v.at[c], csrs_hbm.at[gpos_v.at[c]], sem)
        pltpu.async_copy(dst_v.at[c], csrd_hbm.at[gpos_v.at[c]], sem)
        return cr

    def drain(c, cr):
        pltpu.make_async_copy(src_v.at[c], csrs_hbm.at[gpos_v.at[c]],
                              sem).wait()
        pltpu.make_async_copy(dst_v.at[c], csrd_hbm.at[gpos_v.at[c]],
                              sem).wait()
        return cr

    for g in range(ECH // _GRP):
        lax.fori_loop(g * _GRP, (g + 1) * _GRP, fire, 0)
        lax.fori_loop(g * _GRP, (g + 1) * _GRP, drain, 0)


def _kroute(src3, dst3, gpos3):
    mesh = plsc.VectorSubcoreMesh(core_axis_name="c", subcore_axis_name="s")
    return pl.kernel(
        _kroute_body,
        out_type=[
            jax.ShapeDtypeStruct(((NW + 1) * SEG,), jnp.int32),
            jax.ShapeDtypeStruct(((NW + 1) * SEG,), jnp.int32),
        ],
        mesh=mesh,
        scratch_types=[
            pltpu.VMEM((ECH, 128), jnp.int32),
            pltpu.VMEM((ECH, 128), jnp.int32),
            pltpu.VMEM((ECH, 128), jnp.int32),
            pltpu.SemaphoreType.DMA,
        ],
    )(src3, dst3, gpos3)


def _kcum(deg3):
    return pl.pallas_call(
        _kcum_body,
        in_specs=[pl.BlockSpec((NC, 80, 128), lambda: (0, 0, 0))],
        out_specs=pl.BlockSpec((8, 512), lambda: (0, 0)),
        out_shape=jax.ShapeDtypeStruct((8, 512), jnp.int32),
    )(deg3)


def _kmain_body(x_hbm, csrs_hbm, csrd_hbm, zrow_hbm, spl_hbm, agg_hbm,
                sidx, didx, rv, acc, spl_s, sem):
    cid = lax.axis_index("c")
    sid = lax.axis_index("s")
    w = cid * NS + sid
    lo = w * RPW
    base = w * SEG

    pltpu.sync_copy(zrow_hbm, acc)
    pltpu.sync_copy(spl_hbm, spl_s)
    cnt = _sread(spl_s, 4, w)

    def edge(j, c128_):
        r = didx[pl.ds(j, 16)][0] - lo
        for l in range(8):
            sl = pl.ds(l * 16, 16)
            acc[r, sl] = acc[r, sl] + rv[j, sl]
        return c128_

    def chunk(c, _):
        pltpu.sync_copy(csrs_hbm.at[pl.ds(base + c * 128, 128)], sidx)
        pltpu.sync_copy(csrd_hbm.at[pl.ds(base + c * 128, 144)], didx)
        for l in range(8):
            sl = pl.ds(l * 16, 16)
            v = sidx[sl]
            sidx[sl] = jnp.minimum(jnp.maximum(v, 0), N - 1)
        pltpu.async_copy(x_hbm.at[sidx], rv, sem).wait()
        n = jnp.minimum(cnt - c * 128, 128)
        lax.fori_loop(0, n, edge, c * 128)
        return _

    lax.fori_loop(0, (cnt + 127) // 128, chunk, 0)

    pltpu.sync_copy(acc, agg_hbm.at[pl.ds(lo, RPW)])


def _kmain(x, csrs, csrd, zrow, spl):
    mesh = plsc.VectorSubcoreMesh(core_axis_name="c", subcore_axis_name="s")
    return pl.kernel(
        _kmain_body,
        out_type=jax.ShapeDtypeStruct((NPAD, D), jnp.float32),
        mesh=mesh,
        scratch_types=[
            pltpu.VMEM((128,), jnp.int32),
            pltpu.VMEM((144,), jnp.int32),
            pltpu.VMEM((128, D), jnp.float32),
            pltpu.VMEM((RPW, D), jnp.float32),
            pltpu.VMEM((8, 512), jnp.int32),
            pltpu.SemaphoreType.DMA,
        ],
    )(x, csrs, csrd, zrow, spl)


def _kfix_body(x_hbm, csrs_hbm, csrd_hbm, spl_hbm, fix_hbm,
               sidx, didx, rv, res_v, cur_v, spl_s, sem):
    cid = lax.axis_index("c")
    sid = lax.axis_index("s")
    w = cid * NS + sid

    pltpu.sync_copy(spl_hbm, spl_s)
    rid = _sread(spl_s, 0, w)
    prev = _sread(spl_s, 0, jnp.maximum(w - 1, 0))
    active = (w < NSPL) & (rid >= 0) & ((w == 0) | (rid != prev))

    @pl.when(active)
    def _():
        own = rid // RPW
        base = own * SEG
        cntw = _sread(spl_s, 4, own)
        for l in range(8):
            sl = pl.ds(l * 16, 16)
            res_v[sl] = jnp.zeros((16,), jnp.float32)
            cur_v[sl] = jnp.zeros((16,), jnp.float32)

        def edge(j, m):
            mine = didx[pl.ds(j, 16)][0] == rid

            @pl.when(mine)
            def _():
                flush = m < -1
                for k2 in range(NSPL):
                    flush = flush | (
                        (spl_s[0, pl.ds(16 * k2, 16)][0] == rid)
                        & (spl_s[1, pl.ds(16 * k2, 16)][0] == m))

                @pl.when(flush)
                def _():
                    for l in range(8):
                        sl = pl.ds(l * 16, 16)
                        res_v[sl] = res_v[sl] + cur_v[sl]
                        cur_v[sl] = jnp.zeros((16,), jnp.float32)

                for l in range(8):
                    sl = pl.ds(l * 16, 16)
                    cur_v[sl] = cur_v[sl] + rv[j, sl]

            return m + jnp.where(mine, 1, 0)

        def chunk(c, m):
            pltpu.sync_copy(csrd_hbm.at[pl.ds(base + c * 128, 144)], didx)
            n = jnp.minimum(cntw - c * 128, 128)

            def cscan(j, cm):
                return cm + jnp.where(didx[pl.ds(j, 16)][0] == rid, 1, 0)

            cnt = lax.fori_loop(0, n, cscan, 0)

            @pl.when(cnt > 0)
            def _():
                pltpu.sync_copy(csrs_hbm.at[pl.ds(base + c * 128, 128)],
                                sidx)
                for l in range(8):
                    sl = pl.ds(l * 16, 16)
                    v = sidx[sl]
                    sidx[sl] = jnp.minimum(jnp.maximum(v, 0), N - 1)
                pltpu.async_copy(x_hbm.at[sidx], rv, sem).wait()
                lax.fori_loop(0, n, edge, m)

            return m + cnt

        lax.fori_loop(0, (cntw + 127) // 128, chunk, 0)

        for l in range(8):
            sl = pl.ds(l * 16, 16)
            res_v[sl] = res_v[sl] + cur_v[sl]
        pltpu.sync_copy(res_v, fix_hbm.at[w])


def _kfix(x, csrs, csrd, spl):
    mesh = plsc.VectorSubcoreMesh(core_axis_name="c", subcore_axis_name="s")
    return pl.kernel(
        _kfix_body,
        out_type=jax.ShapeDtypeStruct((NW, D), jnp.float32),
        mesh=mesh,
        scratch_types=[
            pltpu.VMEM((128,), jnp.int32),
            pltpu.VMEM((144,), jnp.int32),
            pltpu.VMEM((128, D), jnp.float32),
            pltpu.VMEM((D,), jnp.float32),
            pltpu.VMEM((D,), jnp.float32),
            pltpu.VMEM((8, 512), jnp.int32),
            pltpu.SemaphoreType.DMA,
        ],
    )(x, csrs, csrd, spl)


def _kscore_body(agg_ref, x_ref, wrel_ref, wroot_ref, clo_ref, deg_ref,
                 scal_ref, spl_ref, fix_ref, score_ref):
    b_rel = scal_ref[0]
    w_close = scal_ref[1]
    w_deg = scal_ref[2]
    w_score = scal_ref[3]
    bias = scal_ref[4]

    d1 = jnp.dot(agg_ref[...], wrel_ref[...])
    d2 = jnp.dot(x_ref[...], wroot_ref[...])
    fixd = jnp.dot(fix_ref[...], wrel_ref[...])

    rowi = lax.broadcasted_iota(jnp.int32, (NPAD, 1), 0)
    for k in range(NSPL):
        rid = spl_ref[0, 16 * k]
        if k == 0:
            valid = rid >= 0
        else:
            valid = (rid >= 0) & (rid != spl_ref[0, 16 * (k - 1)])
        d1 = jnp.where((rowi == rid) & valid, fixd[k:k + 1, 0:1], d1)

    gnn = (d1 + b_rel) + d2
    s1 = jnp.maximum(gnn, 0.0)
    cent = (clo_ref[...] * w_close + deg_ref[...] * w_deg) + bias
    sc = jnp.maximum(s1 * w_score + cent, 0.0)
    sc = jnp.where(rowi < N, sc, 0.0)
    sc = jnp.where(sc == 0.0, 0.0, sc)
    score_ref[...] = sc


def _kscore(agg, x_pad, wrel, wroot, clo2, deg2, scals, spl, fixrows):
    return pl.pallas_call(
        _kscore_body,
        in_specs=[
            pl.BlockSpec((NPAD, D), lambda: (0, 0)),
            pl.BlockSpec((NPAD, D), lambda: (0, 0)),
            pl.BlockSpec((D, 1), lambda: (0, 0)),
            pl.BlockSpec((D, 1), lambda: (0, 0)),
            pl.BlockSpec((NPAD, 1), lambda: (0, 0)),
            pl.BlockSpec((NPAD, 1), lambda: (0, 0)),
            pl.BlockSpec(memory_space=pltpu.SMEM),
            pl.BlockSpec(memory_space=pltpu.SMEM),
            pl.BlockSpec((NW, D), lambda: (0, 0)),
        ],
        out_specs=pl.BlockSpec((NPAD, 1), lambda: (0, 0)),
        out_shape=jax.ShapeDtypeStruct((NPAD, 1), jnp.float32),
    )(agg, x_pad, wrel, wroot, clo2, deg2, scals, spl, fixrows)


def _ksort_body(score_ref, vals_ref, idx_ref):
    score = score_ref[...]
    row = lax.broadcasted_iota(jnp.int32, (SROW, 128), 0)
    col = lax.broadcasted_iota(jnp.int32, (SROW, 128), 1)
    pos = row * 128 + col

    key = lax.bitcast_convert_type(score, jnp.int32)
    idx = pos

    for lk in range(1, 15):
        k = 1 << lk
        asc = (pos & k) == 0
        for lj in range(lk - 1, -1, -1):
            j = 1 << lj
            if j >= 128:
                m = j // 128
                low = (row & m) == 0
                kp = jnp.where(low, pltpu.roll(key, SROW - m, 0),
                               pltpu.roll(key, m, 0))
                ip = jnp.where(low, pltpu.roll(idx, SROW - m, 0),
                               pltpu.roll(idx, m, 0))
            else:
                low = (col & j) == 0
                kp = jnp.where(low, pltpu.roll(key, 128 - j, 1),
                               pltpu.roll(key, j, 1))
                ip = jnp.where(low, pltpu.roll(idx, 128 - j, 1),
                               pltpu.roll(idx, j, 1))
            mp = (key > kp) | ((key == kp) & (idx < ip))
            take_mine = (mp == low) == asc
            key = jnp.where(take_mine, key, kp)
            idx = jnp.where(take_mine, idx, ip)

    vals_ref[...] = lax.bitcast_convert_type(key, jnp.float32)
    idx_ref[...] = idx


def _ksort(score2d):
    return pl.pallas_call(
        _ksort_body,
        in_specs=[pl.BlockSpec((SROW, 128), lambda: (0, 0))],
        out_specs=[
            pl.BlockSpec((SROW, 128), lambda: (0, 0)),
            pl.BlockSpec((SROW, 128), lambda: (0, 0)),
        ],
        out_shape=[
            jax.ShapeDtypeStruct((SROW, 128), jnp.float32),
            jax.ShapeDtypeStruct((SROW, 128), jnp.int32),
        ],
    )(score2d)


def _kgather_body(x_hbm, perm_hbm, rows_hbm, idx_v, rows_v):
    cid = lax.axis_index("c")
    sid = lax.axis_index("s")
    w = cid * NS + sid
    pltpu.sync_copy(perm_hbm.at[w], idx_v)
    for b in range(2):
        pltpu.sync_copy(x_hbm.at[idx_v.at[b]], rows_v.at[b])
    pltpu.sync_copy(rows_v, rows_hbm.at[pl.ds(2 * w, 2)])


def _kgather(x, perm3):
    mesh = plsc.VectorSubcoreMesh(core_axis_name="c", subcore_axis_name="s")
    return pl.kernel(
        _kgather_body,
        out_type=jax.ShapeDtypeStruct((KPAD // 80, 80, D), jnp.float32),
        mesh=mesh,
        scratch_types=[
            pltpu.VMEM((2, 80), jnp.int32),
            pltpu.VMEM((2, 80, D), jnp.float32),
        ],
    )(x, perm3)


def _kscale_body(rows_ref, vals_ref, o_ref):
    o_ref[...] = rows_ref[...] * vals_ref[...]


def _kscale(rows, vals_col):
    tile = 1024
    return pl.pallas_call(
        _kscale_body,
        grid=(KPAD // tile,),
        in_specs=[
            pl.BlockSpec((tile, D), lambda i: (i, 0)),
            pl.BlockSpec((tile, 1), lambda i: (i, 0)),
        ],
        out_specs=pl.BlockSpec((tile, D), lambda i: (i, 0)),
        out_shape=jax.ShapeDtypeStruct((KPAD, D), jnp.float32),
    )(rows, vals_col)


def kernel(x, edge_index, closeness, degree, W_rel, b_rel, W_root,
           w_close, w_deg, w_score, bias):
    src = edge_index[0]
    dst = edge_index[1]

    epad = NW * ECH * 128 - E
    dst_pad = jnp.concatenate([dst, jnp.full((epad,), NPAD, jnp.int32)])
    src_pad = jnp.concatenate([src, jnp.zeros((epad,), jnp.int32)])
    dst3 = dst_pad.reshape(NW, ECH, 128)
    src3 = src_pad.reshape(NW, ECH, 128)
    ones2 = jnp.ones((ECH, 128), jnp.int32)
    zeros_i = jnp.zeros((HN,), jnp.int32)
    hist = _kdeg(dst3, ones2, zeros_i)

    deg3 = hist[:, :NPAD].reshape(NC, 80, 128)
    spl = _kcum(deg3)

    gpos = _kpos(dst_pad.reshape(EROWS, 128))
    csrs, csrd = _kroute(src3, dst3, gpos.reshape(NW, ECH, 128))

    zrow = jnp.zeros((RPW, D), jnp.float32)
    agg = _kmain(x, csrs, csrd, zrow, spl)
    fixrows = _kfix(x, csrs, csrd, spl)

    x_pad = jnp.concatenate([x, jnp.zeros((NPAD - N, D), jnp.float32)])

    def pad_col(a):
        return jnp.concatenate(
            [a, jnp.zeros((NPAD - N,), jnp.float32)]).reshape(NPAD, 1)

    scals = jnp.stack([b_rel[0], w_close[0], w_deg[0], w_score[0], bias[0]])
    score = _kscore(agg, x_pad, W_rel, W_root, pad_col(closeness),
                    pad_col(degree), scals, spl, fixrows)

    score2d = jnp.concatenate(
        [score.reshape(-1),
         jnp.zeros((SROW * 128 - NPAD,), jnp.float32)]).reshape(SROW, 128)
    top_vals2, perm2 = _ksort(score2d)

    perm_flat = perm2.reshape(-1)
    vals_flat = top_vals2.reshape(-1)
    perm3 = perm_flat[:KPAD].reshape(NW, 2, 80)
    rows3 = _kgather(x, perm3)

    rows = rows3.reshape(KPAD, D)
    vals_col = vals_flat[:KPAD].reshape(KPAD, 1)
    x_out = _kscale(rows, vals_col)[:K]

    perm = perm_flat[:K]
    batch_out = jnp.zeros((K,), jnp.int32)
    return (x_out, perm, batch_out)
